# baseline (speedup 1.0000x reference)
import functools

import jax
import jax.numpy as jnp
from jax import lax
from jax.experimental import pallas as pl
from jax.experimental.pallas import tpu as pltpu
from jax.experimental.pallas import tpu_sc as plsc

VOCAB = 100000
D = 768
B = 4
S = 2048
NC = 2
NS = 16
NW = NC * NS
RPW = S // NW
LANES = 16
VECS_PER_ROW = D // LANES


def _body(ids_hbm, tok_hbm, pos_hbm, out_hbm,
          idx_all, pos_v, tok_v, sem_ids, sem_pos, sem_g):
    wid = lax.axis_index("s") * NC + lax.axis_index("c")
    base = wid * RPW

    ph = pltpu.make_async_copy(pos_hbm.at[pl.ds(base, RPW)], pos_v, sem_pos)
    ph.start()
    ih = []
    for b in range(B):
        h = pltpu.make_async_copy(
            ids_hbm.at[b, pl.ds(base, RPW)],
            idx_all.at[pl.ds(b * RPW, RPW)], sem_ids)
        h.start()
        ih.append(h)
    for h in ih:
        h.wait()
    ph.wait()

    def batch_body(b, carry):
        gh = pltpu.make_async_copy(
            tok_hbm.at[idx_all.at[pl.ds(b * RPW, RPW)]], tok_v, sem_g)
        gh.start()
        gh.wait()

        @plsc.parallel_loop(0, RPW)
        def add_row(r):
            pv = pos_v.at[r]
            for j in range(VECS_PER_ROW):
                sl = pl.ds(j * LANES, LANES)
                plsc.addupdate(tok_v.at[r, sl], pv[sl])
        pltpu.sync_copy(tok_v, out_hbm.at[b, pl.ds(base, RPW)])
        return carry

    lax.fori_loop(0, B, batch_body, 0)


@functools.partial(jax.jit, static_argnames=())
def _embed(ids_flat, token_table, position_table):
    mesh = plsc.VectorSubcoreMesh(core_axis_name="c", subcore_axis_name="s")
    run = pl.kernel(
        _body,
        out_type=jax.ShapeDtypeStruct((B, S, D), jnp.float32),
        mesh=mesh,
        scratch_types=[
            pltpu.VMEM((B * RPW,), jnp.int32),
            pltpu.VMEM((RPW, D), jnp.float32),
            pltpu.VMEM((RPW, D), jnp.float32),
            pltpu.SemaphoreType.DMA,
            pltpu.SemaphoreType.DMA,
            pltpu.SemaphoreType.DMA,
        ],
    )
    return run(ids_flat, token_table, position_table)


def kernel(input_ids, token_table, position_table):
    return _embed(input_ids.astype(jnp.int32), token_table, position_table)


# speedup vs baseline: 1.0575x; 1.0575x over previous
import functools

import jax
import jax.numpy as jnp
from jax import lax
from jax.experimental import pallas as pl
from jax.experimental.pallas import tpu as pltpu
from jax.experimental.pallas import tpu_sc as plsc

VOCAB = 100000
D = 768
B = 4
S = 2048
NC = 2
NS = 16
NW = NC * NS
RPW = S // NW
PPC = 16
NCHUNK = RPW // PPC
LANES = 16
HALF = D // 2
VECS_PER_HALF = HALF // LANES


def _body(ids_hbm, tok_hbm, pos_hbm, out_hbm,
          idx_v, pos_v, tok_v, sem_ids, sem_pos, sem_g, sem_w):
    wid = lax.axis_index("s") * NC + lax.axis_index("c")
    base = wid * RPW

    ph = pltpu.make_async_copy(pos_hbm.at[pl.ds(base, RPW)], pos_v, sem_pos)
    ph.start()
    ih = []
    for c in range(NCHUNK):
        for b in range(B):
            h = pltpu.make_async_copy(
                ids_hbm.at[b, pl.ds(base + c * PPC, PPC)],
                idx_v.at[pl.ds(c * PPC * B + b * PPC, PPC)], sem_ids)
            h.start()
            ih.append(h)
    for h in ih:
        h.wait()
    ph.wait()

    def chunk_body(c, carry):
        gh = pltpu.make_async_copy(
            tok_hbm.at[idx_v.at[pl.ds(c * PPC * B, PPC * B)]], tok_v, sem_g)
        gh.start()
        gh.wait()

        def pos_body(p, c2):
            pv = pos_v.at[c * PPC + p]
            for hh in range(2):
                pvec = [pv[pl.ds(hh * HALF + j * LANES, LANES)]
                        for j in range(VECS_PER_HALF)]
                for b in range(B):
                    tv = tok_v.at[b * PPC + p]
                    for j in range(VECS_PER_HALF):
                        sl = pl.ds(hh * HALF + j * LANES, LANES)
                        tv[sl] = tv[sl] + pvec[j]
            return c2

        lax.fori_loop(0, PPC, pos_body, 0)

        whs = []
        for b in range(B):
            wh = pltpu.make_async_copy(
                tok_v.at[pl.ds(b * PPC, PPC)],
                out_hbm.at[b, pl.ds(base + c * PPC, PPC)], sem_w)
            wh.start()
            whs.append(wh)
        for wh in whs:
            wh.wait()
        return carry

    lax.fori_loop(0, NCHUNK, chunk_body, 0)


@functools.partial(jax.jit, static_argnames=())
def _embed(input_ids, token_table, position_table):
    mesh = plsc.VectorSubcoreMesh(core_axis_name="c", subcore_axis_name="s")
    run = pl.kernel(
        _body,
        out_type=jax.ShapeDtypeStruct((B, S, D), jnp.float32),
        mesh=mesh,
        scratch_types=[
            pltpu.VMEM((B * RPW,), jnp.int32),
            pltpu.VMEM((RPW, D), jnp.float32),
            pltpu.VMEM((PPC * B, D), jnp.float32),
            pltpu.SemaphoreType.DMA,
            pltpu.SemaphoreType.DMA,
            pltpu.SemaphoreType.DMA,
            pltpu.SemaphoreType.DMA,
        ],
    )
    return run(input_ids, token_table, position_table)


def kernel(input_ids, token_table, position_table):
    return _embed(input_ids.astype(jnp.int32), token_table, position_table)


# speedup vs baseline: 1.1919x; 1.1271x over previous
import functools

import jax
import jax.numpy as jnp
from jax import lax
from jax.experimental import pallas as pl
from jax.experimental.pallas import tpu as pltpu
from jax.experimental.pallas import tpu_sc as plsc

VOCAB = 100000
D = 768
B = 4
S = 2048
NC = 2
NS = 16
NW = NC * NS
RPW = S // NW
PPC = 16
NCHUNK = RPW // PPC
LANES = 16
HALF = D // 2
VECS_PER_HALF = HALF // LANES


def _body(ids_hbm, tok_hbm, pos_hbm, out_hbm,
          idx_v, pos0, pos1, tok0, tok1,
          sem_ids, sp0, sp1, sg0, sg1, sw0, sw1):
    wid = lax.axis_index("s") * NC + lax.axis_index("c")
    base = wid * RPW

    poss = (pos0, pos1)
    toks = (tok0, tok1)
    psems = (sp0, sp1)
    gsems = (sg0, sg1)
    wsems = (sw0, sw1)

    ih = []
    for c in range(NCHUNK):
        for b in range(B):
            h = pltpu.make_async_copy(
                ids_hbm.at[b, pl.ds(base + c * PPC, PPC)],
                idx_v.at[pl.ds(c * PPC * B + b * PPC, PPC)], sem_ids)
            h.start()
            ih.append(h)

    def start_pos(c):
        k = c % 2
        h = pltpu.make_async_copy(
            pos_hbm.at[pl.ds(base + c * PPC, PPC)], poss[k], psems[k])
        h.start()
        return h

    ph = [start_pos(0), start_pos(1)]
    for h in ih:
        h.wait()

    gh = [None, None]

    def start_gather(c):
        k = c % 2
        gh[k] = pltpu.make_async_copy(
            tok_hbm.at[idx_v.at[pl.ds(c * PPC * B, PPC * B)]],
            toks[k], gsems[k])
        gh[k].start()

    start_gather(0)
    start_gather(1)

    wh = [None, None]
    for c in range(NCHUNK):
        k = c % 2
        gh[k].wait()
        ph[k].wait()
        tok_v = toks[k]
        pos_v = poss[k]

        def pos_body(p, c2, tok_v=tok_v, pos_v=pos_v):
            pv = pos_v.at[p]
            for hh in range(2):
                pvec = [pv[pl.ds(hh * HALF + j * LANES, LANES)]
                        for j in range(VECS_PER_HALF)]
                for b in range(B):
                    tv = tok_v.at[b * PPC + p]
                    for j in range(VECS_PER_HALF):
                        sl = pl.ds(hh * HALF + j * LANES, LANES)
                        tv[sl] = tv[sl] + pvec[j]
            return c2

        lax.fori_loop(0, PPC, pos_body, 0)

        wh[k] = []
        for b in range(B):
            w = pltpu.make_async_copy(
                tok_v.at[pl.ds(b * PPC, PPC)],
                out_hbm.at[b, pl.ds(base + c * PPC, PPC)], wsems[k])
            w.start()
            wh[k].append(w)

        if c + 2 < NCHUNK:
            for w in wh[k]:
                w.wait()
            wh[k] = None
            ph[k] = start_pos(c + 2)
            start_gather(c + 2)

    for k in range(2):
        if wh[k] is not None:
            for w in wh[k]:
                w.wait()


@functools.partial(jax.jit, static_argnames=())
def _embed(input_ids, token_table, position_table):
    mesh = plsc.VectorSubcoreMesh(core_axis_name="c", subcore_axis_name="s")
    run = pl.kernel(
        _body,
        out_type=jax.ShapeDtypeStruct((B, S, D), jnp.float32),
        mesh=mesh,
        scratch_types=[
            pltpu.VMEM((B * RPW,), jnp.int32),
            pltpu.VMEM((PPC, D), jnp.float32),
            pltpu.VMEM((PPC, D), jnp.float32),
            pltpu.VMEM((PPC * B, D), jnp.float32),
            pltpu.VMEM((PPC * B, D), jnp.float32),
            pltpu.SemaphoreType.DMA,
            pltpu.SemaphoreType.DMA,
            pltpu.SemaphoreType.DMA,
            pltpu.SemaphoreType.DMA,
            pltpu.SemaphoreType.DMA,
            pltpu.SemaphoreType.DMA,
            pltpu.SemaphoreType.DMA,
        ],
    )
    return run(input_ids, token_table, position_table)


def kernel(input_ids, token_table, position_table):
    return _embed(input_ids.astype(jnp.int32), token_table, position_table)
